# baseline (speedup 1.0000x reference)
import jax
import jax.numpy as jnp
import numpy as np
from jax import lax
from jax.experimental import pallas as pl
from jax.experimental.pallas import tpu as pltpu
from jax.experimental.pallas import tpu_sc as plsc

N = 10000
E = 320000
H = 128
HEADS = 4
DH = 32
T = 2
SCALE = float(np.sqrt(DH))

NC = 2
NS = 16
NW = NC * NS
EPW = E // NW
B = 64
NB = EPW // B
TAIL = EPW - NB * B

SBASE = 10240
AROWS = SBASE + 320
DUMP = SBASE + 316
ZR = 664
ZR_LAST = AROWS - 15 * ZR

NPAD = 320 * DH

_f32 = jnp.float32


def _prep_body(x_ref, wq_ref, bq_ref, wk_ref, bk_ref, wv_ref, bv_ref,
               qt_ref, kv_ref):
    xb = x_ref[...]
    q = jnp.dot(xb, wq_ref[...].T, preferred_element_type=_f32) + bq_ref[...]
    k = jnp.dot(xb, wk_ref[...].T, preferred_element_type=_f32) + bk_ref[...]
    v = jnp.dot(xb, wv_ref[...].T, preferred_element_type=_f32) + bv_ref[...]
    qt_ref[...] = q * (1.0 / SCALE)
    kv_ref[...] = jnp.concatenate([k, v], axis=1)


def _prep(x, Wq, bq, Wk, bk, Wv, bv):
    rows = N // 10
    return pl.pallas_call(
        _prep_body,
        grid=(10,),
        in_specs=[
            pl.BlockSpec((rows, H), lambda i: (i, 0)),
            pl.BlockSpec((H, H), lambda i: (0, 0)),
            pl.BlockSpec((H,), lambda i: (0,)),
            pl.BlockSpec((H, H), lambda i: (0, 0)),
            pl.BlockSpec((H,), lambda i: (0,)),
            pl.BlockSpec((H, H), lambda i: (0, 0)),
            pl.BlockSpec((H,), lambda i: (0,)),
        ],
        out_specs=[
            pl.BlockSpec((rows, H), lambda i: (i, 0)),
            pl.BlockSpec((rows, 2 * H), lambda i: (i, 0)),
        ],
        out_shape=[
            jax.ShapeDtypeStruct((N, H), _f32),
            jax.ShapeDtypeStruct((N, 2 * H), _f32),
        ],
    )(x, Wq, bq, Wk, bk, Wv, bv)


def _bcast_gather(x, idx):
    dn = lax.GatherDimensionNumbers(
        offset_dims=(), collapsed_slice_dims=(0,), start_index_map=(0,))
    return lax.gather(x, idx[:, None], dn, slice_sizes=(1,),
                      mode=lax.GatherScatterMode.PROMISE_IN_BOUNDS)


def _edge_kernel_body(qt_hbm, kv_hbm, src_hbm, dst_hbm, p_hbm, sk_hbm, sv_hbm,
                      out_hbm,
                      srcv, dstv, dstgv, pv, skv, svv, qtv, kvv, cvv, cvs,
                      acc, seml, semg, sems):
    cid = lax.axis_index("c")
    sid = lax.axis_index("s")
    wid = sid * NC + cid
    lane = lax.iota(jnp.int32, 16)
    zero16 = jnp.zeros((16,), _f32)

    pltpu.sync_copy(sk_hbm, skv)
    pltpu.sync_copy(sv_hbm, svv)
    skc = [skv[pl.ds(c * 16, 16)] for c in range(8)]
    svc = [svv[pl.ds(c * 16, 16)] for c in range(8)]

    def _zrow(i, _):
        cvv[i // 8, pl.ds((i % 8) * 16, 16)] = zero16
        cvs[i // 8, pl.ds((i % 8) * 16, 16)] = zero16
        return 0
    lax.fori_loop(0, B * (H // 16), _zrow, 0)

    z0 = sid * ZR

    @pl.when(sid == NS - 1)
    def _():
        for j in range(ZR_LAST // B):
            pltpu.sync_copy(cvv, acc.at[pl.ds(z0 + j * B, B)])
        pltpu.sync_copy(cvv.at[pl.ds(0, ZR_LAST % B)],
                        acc.at[pl.ds(z0 + (ZR_LAST // B) * B, ZR_LAST % B)])

    @pl.when(sid != NS - 1)
    def _():
        for j in range(ZR // B):
            pltpu.sync_copy(cvv, acc.at[pl.ds(z0 + j * B, B)])
        pltpu.sync_copy(cvv.at[pl.ds(0, ZR % B)],
                        acc.at[pl.ds(z0 + (ZR // B) * B, ZR % B)])

    plsc.subcore_barrier()

    lx8 = jnp.bitwise_xor(lane, 8)
    lx4 = jnp.bitwise_xor(lane, 4)
    lx2 = jnp.bitwise_xor(lane, 2)
    lx1 = jnp.bitwise_xor(lane, 1)
    lane4 = (lane * 4) & 15
    laneh4 = jnp.where(lane < HEADS, lane, 0)
    lmasks = [lane // 4 == hh for hh in range(HEADS)]
    exbi = [jnp.full((16,), 4 * hh, jnp.int32) for hh in range(HEADS)]

    def _group(slot, g):
        pch = pv[slot, pl.ds(g * 16, 16)]
        dch = dstv[slot, pl.ds(g * 16, 16)]

        def _e(i, _):
            e = g * 16 + i
            sub = jnp.full((16,), i, jnp.int32)
            pe = _bcast_gather(pch, sub)
            dste = _bcast_gather(dch, sub)
            us = []
            for hh in range(HEADS):
                c0 = hh * DH
                kj0 = kvv[e, pl.ds(c0, 16)] + pe * skc[2 * hh]
                kj1 = kvv[e, pl.ds(c0 + 16, 16)] + pe * skc[2 * hh + 1]
                s = (qtv[e, pl.ds(c0, 16)] * kj0
                     + qtv[e, pl.ds(c0 + 16, 16)] * kj1)
                t = s + _bcast_gather(s, lx8)
                us.append(t + _bcast_gather(t, lx4))
            w = jnp.where(lmasks[0], us[0],
                          jnp.where(lmasks[1], us[1],
                                    jnp.where(lmasks[2], us[2], us[3])))
            x1 = w + _bcast_gather(w, lx2)
            alpha = x1 + _bcast_gather(x1, lx1)
            exr = jnp.exp(alpha)
            exl = _bcast_gather(exr, lane4)
            for j in range(H // 16):
                cvs[e, pl.ds(j * 16, 16)] = zero16
            colv = (dste % 32) * HEADS + laneh4
            plsc.store_scatter(cvs, [jnp.full((16,), e, jnp.int32), colv],
                               exl, mask=lane < HEADS)
            for hh in range(HEADS):
                exb = _bcast_gather(exr, exbi[hh])
                c0 = hh * DH
                vj0 = kvv[e, pl.ds(H + c0, 16)] + pe * svc[2 * hh]
                vj1 = kvv[e, pl.ds(H + c0 + 16, 16)] + pe * svc[2 * hh + 1]
                cvv[e, pl.ds(c0, 16)] = vj0 * exb
                cvv[e, pl.ds(c0 + 16, 16)] = vj1 * exb
            return 0
        lax.fori_loop(0, 16, _e, 0)
        return 0

    def _issue_idx(slot, blk):
        base = wid * EPW + blk * B
        pltpu.async_copy(src_hbm.at[pl.ds(base, B)], srcv.at[slot], seml)
        pltpu.async_copy(dst_hbm.at[pl.ds(base, B)], dstv.at[slot], seml)
        pltpu.async_copy(p_hbm.at[pl.ds(base, B)], pv.at[slot], seml)

    def _wait_idx():
        pltpu.make_async_copy(src_hbm.at[pl.ds(0, B)], srcv.at[0],
                              seml).wait()
        pltpu.make_async_copy(dst_hbm.at[pl.ds(0, B)], dstv.at[0],
                              seml).wait()
        pltpu.make_async_copy(p_hbm.at[pl.ds(0, B)], pv.at[0], seml).wait()

    def _mk_dstg(slot):
        for j in range(B // 16):
            dstgv[slot, pl.ds(j * 16, 16)] = jnp.minimum(
                dstv[slot, pl.ds(j * 16, 16)] // 32 + SBASE, DUMP)

    def _issue_gathers(slot):
        pltpu.async_copy(kv_hbm.at[srcv.at[slot]], kvv, semg)
        pltpu.async_copy(qt_hbm.at[dstv.at[slot]], qtv, semg)

    def _wait_gathers():
        pltpu.make_async_copy(kv_hbm.at[srcv.at[0]], kvv, semg).wait()
        pltpu.make_async_copy(qt_hbm.at[dstv.at[0]], qtv, semg).wait()

    def _issue_scatters(slot):
        pltpu.async_copy(cvv, acc.at[dstv.at[slot]], sems, add=True)
        pltpu.async_copy(cvs, acc.at[dstgv.at[slot]], sems, add=True)

    def _wait_scatters():
        pltpu.make_async_copy(cvv, acc.at[dstv.at[0]], sems).wait()
        pltpu.make_async_copy(cvs, acc.at[dstgv.at[0]], sems).wait()

    _issue_idx(0, 0)
    _wait_idx()
    _mk_dstg(0)
    _issue_gathers(0)

    def _blk(blk, _):
        slot = blk % 2
        nslot = (blk + 1) % 2

        @pl.when(blk > 0)
        def _():
            _wait_scatters()

        @pl.when(blk + 1 < NB)
        def _():
            _issue_idx(nslot, blk + 1)

        _wait_gathers()
        lax.fori_loop(0, B // 16, lambda g, c: _group(slot, g), 0)
        _issue_scatters(slot)

        @pl.when(blk + 1 < NB)
        def _():
            _wait_idx()
            _mk_dstg(nslot)
            _issue_gathers(nslot)
        return 0
    lax.fori_loop(0, NB, _blk, 0)
    _wait_scatters()

    def _ztail(i, _):
        r = TAIL + i // 8
        cvv[r, pl.ds((i % 8) * 16, 16)] = zero16
        cvs[r, pl.ds((i % 8) * 16, 16)] = zero16
        return 0
    lax.fori_loop(0, (B - TAIL) * (H // 16), _ztail, 0)
    for j in range(B // 16):
        dstv[0, pl.ds(j * 16, 16)] = jnp.full((16,), DUMP, jnp.int32)
        srcv[0, pl.ds(j * 16, 16)] = jnp.zeros((16,), jnp.int32)
    tb = wid * EPW + NB * B
    pltpu.sync_copy(src_hbm.at[pl.ds(tb, TAIL)], srcv.at[0, pl.ds(0, TAIL)])
    pltpu.sync_copy(dst_hbm.at[pl.ds(tb, TAIL)], dstv.at[0, pl.ds(0, TAIL)])
    pltpu.sync_copy(p_hbm.at[pl.ds(tb, TAIL)], pv.at[0, pl.ds(0, TAIL)])
    _mk_dstg(0)
    _issue_gathers(0)
    _wait_gathers()
    _group(0, 0)
    _issue_scatters(0)
    _wait_scatters()

    plsc.subcore_barrier()

    @pl.when(sid == NS - 1)
    def _():
        pltpu.sync_copy(acc.at[pl.ds(z0, ZR_LAST)],
                        out_hbm.at[cid, pl.ds(z0, ZR_LAST)])

    @pl.when(sid != NS - 1)
    def _():
        pltpu.sync_copy(acc.at[pl.ds(z0, ZR)],
                        out_hbm.at[cid, pl.ds(z0, ZR)])


def _edge(qt, kv, src, dst, p, sk, sv):
    mesh = plsc.VectorSubcoreMesh(core_axis_name="c", subcore_axis_name="s")
    fn = pl.kernel(
        _edge_kernel_body,
        out_type=jax.ShapeDtypeStruct((NC, AROWS, H), _f32),
        mesh=mesh,
        compiler_params=pltpu.CompilerParams(needs_layout_passes=False),
        scratch_types=[
            pltpu.VMEM((2, B), jnp.int32),
            pltpu.VMEM((2, B), jnp.int32),
            pltpu.VMEM((2, B), jnp.int32),
            pltpu.VMEM((2, B), _f32),
            pltpu.VMEM((H,), _f32),
            pltpu.VMEM((H,), _f32),
            pltpu.VMEM((B, H), _f32),
            pltpu.VMEM((B, 2 * H), _f32),
            pltpu.VMEM((B, H), _f32),
            pltpu.VMEM((B, H), _f32),
            pltpu.VMEM_SHARED((AROWS, H), _f32),
            pltpu.SemaphoreType.DMA,
            pltpu.SemaphoreType.DMA,
            pltpu.SemaphoreType.DMA,
        ],
    )
    return fn(qt, kv, src, dst, p, sk, sv)


def _expand_body(stats_ref, out_ref):
    dsum = stats_ref[0] + stats_ref[1]
    jj = lax.broadcasted_iota(jnp.int32, (H, DH, H), 0)
    kk = lax.broadcasted_iota(jnp.int32, (H, DH, H), 1)
    cc = lax.broadcasted_iota(jnp.int32, (H, DH, H), 2)
    p = (jj == HEADS * kk + cc // DH).astype(_f32)
    full = lax.dot_general(dsum, p, (((1,), (0,)), ((), ())),
                           preferred_element_type=_f32)
    out_ref[...] = full.reshape(NPAD, H)


def _expand_den(parts):
    return pl.pallas_call(
        _expand_body,
        grid=(1,),
        in_specs=[pl.BlockSpec((NC, 320, H), lambda i: (0, SBASE // 320, 0))],
        out_specs=pl.BlockSpec((NPAD, H), lambda i: (0, 0)),
        out_shape=jax.ShapeDtypeStruct((NPAD, H), _f32),
    )(parts)


def _ln(x, g, b, eps=1e-12):
    u = x.mean(-1, keepdims=True)
    s = ((x - u) ** 2).mean(-1, keepdims=True)
    return g * (x - u) / jnp.sqrt(s + eps) + b


def _post_body(parts_ref, dens_ref, x_ref, h_ref, wao_ref, bao_ref, g1_ref,
               b1_ref, wi_ref, bi_ref, wo_ref, bo_ref, g2_ref, b2_ref,
               wih_ref, whh_ref, bih_ref, bhh_ref, g3_ref, b3_ref,
               xo_ref, ho_ref):
    aggv = parts_ref[0] + parts_ref[1]
    den = dens_ref[...] + 1e-16
    agg = aggv / den
    x = x_ref[...]
    h = h_ref[...]
    attn = _ln(jnp.dot(agg, wao_ref[...].T, preferred_element_type=_f32)
               + bao_ref[...] + x, g1_ref[...], b1_ref[...])
    inter = jax.nn.gelu(jnp.dot(attn, wi_ref[...].T,
                                preferred_element_type=_f32) + bi_ref[...])
    m = _ln(jnp.dot(inter, wo_ref[...].T, preferred_element_type=_f32)
            + bo_ref[...] + attn, g2_ref[...], b2_ref[...])
    gi = jnp.dot(m, wih_ref[...].T, preferred_element_type=_f32) + bih_ref[...]
    gh = jnp.dot(h, whh_ref[...].T, preferred_element_type=_f32) + bhh_ref[...]
    r = jax.nn.sigmoid(gi[:, 0:H] + gh[:, 0:H])
    z = jax.nn.sigmoid(gi[:, H:2 * H] + gh[:, H:2 * H])
    n = jnp.tanh(gi[:, 2 * H:3 * H] + r * gh[:, 2 * H:3 * H])
    hn = (1.0 - z) * n + z * h
    ho_ref[...] = hn
    xo_ref[...] = _ln(hn, g3_ref[...], b3_ref[...])


def _post(parts, dens, x, h, Wao, bao, g1, b1, Wi, bi, Wo, bo, g2, b2,
          W_ih, W_hh, b_ih, b_hh, g3, b3):
    rows = N // 10
    full = lambda shape: pl.BlockSpec(shape, lambda i: (0,) * len(shape))
    vec = full((H,))
    return pl.pallas_call(
        _post_body,
        grid=(10,),
        in_specs=[
            pl.BlockSpec((NC, rows, H), lambda i: (0, i, 0)),
            pl.BlockSpec((rows, H), lambda i: (i, 0)),
            pl.BlockSpec((rows, H), lambda i: (i, 0)),
            pl.BlockSpec((rows, H), lambda i: (i, 0)),
            full((H, H)), vec,
            vec, vec,
            full((4 * H, H)), full((4 * H,)),
            full((H, 4 * H)), vec,
            vec, vec,
            full((3 * H, H)), full((3 * H, H)),
            full((3 * H,)), full((3 * H,)),
            vec, vec,
        ],
        out_specs=[
            pl.BlockSpec((rows, H), lambda i: (i, 0)),
            pl.BlockSpec((rows, H), lambda i: (i, 0)),
        ],
        out_shape=[
            jax.ShapeDtypeStruct((N, H), _f32),
            jax.ShapeDtypeStruct((N, H), _f32),
        ],
    )(parts, dens, x, h, Wao, bao, g1, b1, Wi, bi, Wo, bo, g2, b2,
      W_ih, W_hh, b_ih, b_hh, g3, b3)


def kernel(x, edge_index, edge_attr, Wq, bq, Wk, bk, Wv, bv, Wao, bao, g1, b1,
           Wi, bi, Wo, bo, g2, b2, W_ih, W_hh, b_ih, b_hh, g3, b3):
    src = edge_index[0]
    dst = edge_index[1]
    sk = jnp.sum(Wk, axis=1)
    sv = jnp.sum(Wv, axis=1)
    h = x
    for _ in range(T):
        qt, kv = _prep(x, Wq, bq, Wk, bk, Wv, bv)
        parts = _edge(qt, kv, src, dst, edge_attr, sk, sv)
        dens = _expand_den(parts)
        x, h = _post(parts, dens, x, h, Wao, bao, g1, b1, Wi, bi, Wo, bo,
                     g2, b2, W_ih, W_hh, b_ih, b_hh, g3, b3)
    return x


# speedup vs baseline: 1.2007x; 1.2007x over previous
import jax
import jax.numpy as jnp
import numpy as np
from jax import lax
from jax.experimental import pallas as pl
from jax.experimental.pallas import tpu as pltpu
from jax.experimental.pallas import tpu_sc as plsc

N = 10000
E = 320000
H = 128
HEADS = 4
DH = 32
T = 2
SCALE = float(np.sqrt(DH))

NC = 2
NS = 16
NW = NC * NS
EPW = E // NW
B = 32
NB = EPW // B
TAIL = EPW - NB * B

SBASE = 10240
AROWS = SBASE + 320
DUMP = SBASE + 316
ZR = 664
ZR_LAST = AROWS - 15 * ZR

NPAD = 320 * DH

_f32 = jnp.float32


def _prep_body(x_ref, wq_ref, bq_ref, wk_ref, bk_ref, wv_ref, bv_ref,
               qt_ref, kv_ref):
    xb = x_ref[...]
    q = jnp.dot(xb, wq_ref[...].T, preferred_element_type=_f32) + bq_ref[...]
    k = jnp.dot(xb, wk_ref[...].T, preferred_element_type=_f32) + bk_ref[...]
    v = jnp.dot(xb, wv_ref[...].T, preferred_element_type=_f32) + bv_ref[...]
    qt_ref[...] = q * (1.0 / SCALE)
    kv_ref[...] = jnp.concatenate([k, v], axis=1)


def _prep(x, Wq, bq, Wk, bk, Wv, bv):
    rows = N // 10
    return pl.pallas_call(
        _prep_body,
        grid=(10,),
        in_specs=[
            pl.BlockSpec((rows, H), lambda i: (i, 0)),
            pl.BlockSpec((H, H), lambda i: (0, 0)),
            pl.BlockSpec((H,), lambda i: (0,)),
            pl.BlockSpec((H, H), lambda i: (0, 0)),
            pl.BlockSpec((H,), lambda i: (0,)),
            pl.BlockSpec((H, H), lambda i: (0, 0)),
            pl.BlockSpec((H,), lambda i: (0,)),
        ],
        out_specs=[
            pl.BlockSpec((rows, H), lambda i: (i, 0)),
            pl.BlockSpec((rows, 2 * H), lambda i: (i, 0)),
        ],
        out_shape=[
            jax.ShapeDtypeStruct((N, H), _f32),
            jax.ShapeDtypeStruct((N, 2 * H), _f32),
        ],
    )(x, Wq, bq, Wk, bk, Wv, bv)


def _bcast_gather(x, idx):
    dn = lax.GatherDimensionNumbers(
        offset_dims=(), collapsed_slice_dims=(0,), start_index_map=(0,))
    return lax.gather(x, idx[:, None], dn, slice_sizes=(1,),
                      mode=lax.GatherScatterMode.PROMISE_IN_BOUNDS)


def _edge_kernel_body(qt_hbm, kv_hbm, src_hbm, dst_hbm, p_hbm, sk_hbm, sv_hbm,
                      out_hbm,
                      srcv, dstv, dstgv, pv, cpv, cdv, sdstg, skv, svv,
                      qtv, kvv, cvv, cvs,
                      acc, seml0, seml1, semg, sems):
    cid = lax.axis_index("c")
    sid = lax.axis_index("s")
    wid = sid * NC + cid
    lane = lax.iota(jnp.int32, 16)
    zero16 = jnp.zeros((16,), _f32)

    pltpu.sync_copy(sk_hbm, skv)
    pltpu.sync_copy(sv_hbm, svv)
    skc = [skv[pl.ds(c * 16, 16)] for c in range(8)]
    svc = [svv[pl.ds(c * 16, 16)] for c in range(8)]

    def _zrow(i, _):
        cvv[i // 8, pl.ds((i % 8) * 16, 16)] = zero16
        cvs[i // 8, pl.ds((i % 8) * 16, 16)] = zero16
        return 0
    lax.fori_loop(0, B * (H // 16), _zrow, 0)

    z0 = sid * ZR

    @pl.when(sid == NS - 1)
    def _():
        for j in range(ZR_LAST // B):
            pltpu.sync_copy(cvv, acc.at[pl.ds(z0 + j * B, B)])
        pltpu.sync_copy(cvv.at[pl.ds(0, ZR_LAST % B)],
                        acc.at[pl.ds(z0 + (ZR_LAST // B) * B, ZR_LAST % B)])

    @pl.when(sid != NS - 1)
    def _():
        for j in range(ZR // B):
            pltpu.sync_copy(cvv, acc.at[pl.ds(z0 + j * B, B)])
        pltpu.sync_copy(cvv.at[pl.ds(0, ZR % B)],
                        acc.at[pl.ds(z0 + (ZR // B) * B, ZR % B)])

    plsc.subcore_barrier()

    lx8 = jnp.bitwise_xor(lane, 8)
    lx4 = jnp.bitwise_xor(lane, 4)
    lx2 = jnp.bitwise_xor(lane, 2)
    lx1 = jnp.bitwise_xor(lane, 1)
    lane4 = (lane * 4) & 15
    laneh4 = jnp.where(lane < HEADS, lane, 0)
    lmasks = [lane // 4 == hh for hh in range(HEADS)]
    exbi = [jnp.full((16,), 4 * hh, jnp.int32) for hh in range(HEADS)]

    def _group(gslot, g):
        pch = cpv[pl.ds(g * 16, 16)]
        dch = cdv[pl.ds(g * 16, 16)]

        def _e(i, _):
            e = g * 16 + i
            sub = jnp.full((16,), i, jnp.int32)
            pe = _bcast_gather(pch, sub)
            dste = _bcast_gather(dch, sub)
            us = []
            for hh in range(HEADS):
                c0 = hh * DH
                kj0 = kvv[gslot, e, pl.ds(c0, 16)] + pe * skc[2 * hh]
                kj1 = kvv[gslot, e, pl.ds(c0 + 16, 16)] + pe * skc[2 * hh + 1]
                s = (qtv[gslot, e, pl.ds(c0, 16)] * kj0
                     + qtv[gslot, e, pl.ds(c0 + 16, 16)] * kj1)
                t = s + _bcast_gather(s, lx8)
                us.append(t + _bcast_gather(t, lx4))
            w = jnp.where(lmasks[0], us[0],
                          jnp.where(lmasks[1], us[1],
                                    jnp.where(lmasks[2], us[2], us[3])))
            x1 = w + _bcast_gather(w, lx2)
            alpha = x1 + _bcast_gather(x1, lx1)
            exr = jnp.exp(alpha)
            exl = _bcast_gather(exr, lane4)
            for j in range(H // 16):
                cvs[e, pl.ds(j * 16, 16)] = zero16
            colv = (dste % 32) * HEADS + laneh4
            plsc.store_scatter(cvs, [jnp.full((16,), e, jnp.int32), colv],
                               exl, mask=lane < HEADS)
            for hh in range(HEADS):
                exb = _bcast_gather(exr, exbi[hh])
                c0 = hh * DH
                vj0 = kvv[gslot, e, pl.ds(H + c0, 16)] + pe * svc[2 * hh]
                vj1 = kvv[gslot, e, pl.ds(H + c0 + 16, 16)] + pe * svc[2 * hh + 1]
                cvv[e, pl.ds(c0, 16)] = vj0 * exb
                cvv[e, pl.ds(c0 + 16, 16)] = vj1 * exb
            return 0
        lax.fori_loop(0, 16, _e, 0)
        return 0

    def _issue_idx(slot, blk, sem):
        base = wid * EPW + blk * B
        pltpu.async_copy(src_hbm.at[pl.ds(base, B)], srcv.at[slot], sem)
        pltpu.async_copy(dst_hbm.at[pl.ds(base, B)], dstv.at[slot], sem)
        pltpu.async_copy(p_hbm.at[pl.ds(base, B)], pv.at[slot], sem)

    def _wait_idx(sem):
        pltpu.make_async_copy(src_hbm.at[pl.ds(0, B)], srcv.at[0], sem).wait()
        pltpu.make_async_copy(dst_hbm.at[pl.ds(0, B)], dstv.at[0], sem).wait()
        pltpu.make_async_copy(p_hbm.at[pl.ds(0, B)], pv.at[0], sem).wait()

    def _mk_dstg(slot):
        for j in range(B // 16):
            dstgv[slot, pl.ds(j * 16, 16)] = jnp.minimum(
                dstv[slot, pl.ds(j * 16, 16)] // 32 + SBASE, DUMP)

    def _snap_idx(slot):
        for j in range(B // 16):
            cpv[pl.ds(j * 16, 16)] = pv[slot, pl.ds(j * 16, 16)]
            cdv[pl.ds(j * 16, 16)] = dstv[slot, pl.ds(j * 16, 16)]
            sdstg[pl.ds(j * 16, 16)] = dstgv[slot, pl.ds(j * 16, 16)]

    def _issue_gathers(slot):
        pltpu.async_copy(kv_hbm.at[srcv.at[slot]], kvv.at[slot], semg)
        pltpu.async_copy(qt_hbm.at[dstv.at[slot]], qtv.at[slot], semg)

    def _wait_gathers():
        pltpu.make_async_copy(kv_hbm.at[srcv.at[0]], kvv.at[0], semg).wait()
        pltpu.make_async_copy(qt_hbm.at[dstv.at[0]], qtv.at[0], semg).wait()

    def _issue_scatters():
        pltpu.async_copy(cvv, acc.at[cdv], sems, add=True)
        pltpu.async_copy(cvs, acc.at[sdstg], sems, add=True)

    def _wait_scatters():
        pltpu.make_async_copy(cvv, acc.at[cdv], sems).wait()
        pltpu.make_async_copy(cvs, acc.at[sdstg], sems).wait()

    _issue_idx(0, 0, seml0)
    _wait_idx(seml0)
    _mk_dstg(0)
    _issue_gathers(0)
    _issue_idx(1, 1, seml1)

    def _blk(blk, _):
        slot = blk % 2
        nslot = (blk + 1) % 2

        @pl.when(blk > 0)
        def _():
            _wait_scatters()
        _wait_gathers()
        _snap_idx(slot)

        @pl.when(blk + 2 < NB)
        def _():
            @pl.when(slot == 0)
            def _():
                _issue_idx(slot, blk + 2, seml0)

            @pl.when(slot == 1)
            def _():
                _issue_idx(slot, blk + 2, seml1)

        @pl.when(blk + 1 < NB)
        def _():
            @pl.when(nslot == 0)
            def _():
                _wait_idx(seml0)

            @pl.when(nslot == 1)
            def _():
                _wait_idx(seml1)
            _mk_dstg(nslot)
            _issue_gathers(nslot)

        lax.fori_loop(0, B // 16, lambda g, c: _group(slot, g), 0)
        _issue_scatters()
        return 0
    lax.fori_loop(0, NB, _blk, 0)
    _wait_scatters()

    def _ztail(i, _):
        r = TAIL + i // 8
        cvv[r, pl.ds((i % 8) * 16, 16)] = zero16
        cvs[r, pl.ds((i % 8) * 16, 16)] = zero16
        return 0
    lax.fori_loop(0, (B - TAIL) * (H // 16), _ztail, 0)
    for j in range(B // 16):
        dstv[0, pl.ds(j * 16, 16)] = jnp.full((16,), DUMP, jnp.int32)
        srcv[0, pl.ds(j * 16, 16)] = jnp.zeros((16,), jnp.int32)
    tb = wid * EPW + NB * B
    pltpu.sync_copy(src_hbm.at[pl.ds(tb, TAIL)], srcv.at[0, pl.ds(0, TAIL)])
    pltpu.sync_copy(dst_hbm.at[pl.ds(tb, TAIL)], dstv.at[0, pl.ds(0, TAIL)])
    pltpu.sync_copy(p_hbm.at[pl.ds(tb, TAIL)], pv.at[0, pl.ds(0, TAIL)])
    _mk_dstg(0)
    _snap_idx(0)
    _issue_gathers(0)
    _wait_gathers()
    _group(0, 0)
    _issue_scatters()
    _wait_scatters()

    plsc.subcore_barrier()

    @pl.when(sid == NS - 1)
    def _():
        pltpu.sync_copy(acc.at[pl.ds(z0, ZR_LAST)],
                        out_hbm.at[cid, pl.ds(z0, ZR_LAST)])

    @pl.when(sid != NS - 1)
    def _():
        pltpu.sync_copy(acc.at[pl.ds(z0, ZR)],
                        out_hbm.at[cid, pl.ds(z0, ZR)])


def _edge(qt, kv, src, dst, p, sk, sv):
    mesh = plsc.VectorSubcoreMesh(core_axis_name="c", subcore_axis_name="s")
    fn = pl.kernel(
        _edge_kernel_body,
        out_type=jax.ShapeDtypeStruct((NC, AROWS, H), _f32),
        mesh=mesh,
        compiler_params=pltpu.CompilerParams(needs_layout_passes=False),
        scratch_types=[
            pltpu.VMEM((2, B), jnp.int32),
            pltpu.VMEM((2, B), jnp.int32),
            pltpu.VMEM((2, B), jnp.int32),
            pltpu.VMEM((2, B), _f32),
            pltpu.VMEM((B,), _f32),
            pltpu.VMEM((B,), jnp.int32),
            pltpu.VMEM((B,), jnp.int32),
            pltpu.VMEM((H,), _f32),
            pltpu.VMEM((H,), _f32),
            pltpu.VMEM((2, B, H), _f32),
            pltpu.VMEM((2, B, 2 * H), _f32),
            pltpu.VMEM((B, H), _f32),
            pltpu.VMEM((B, H), _f32),
            pltpu.VMEM_SHARED((AROWS, H), _f32),
            pltpu.SemaphoreType.DMA,
            pltpu.SemaphoreType.DMA,
            pltpu.SemaphoreType.DMA,
            pltpu.SemaphoreType.DMA,
        ],
    )
    return fn(qt, kv, src, dst, p, sk, sv)


def _expand_body(stats_ref, out_ref):
    dsum = stats_ref[0] + stats_ref[1]
    jj = lax.broadcasted_iota(jnp.int32, (H, DH, H), 0)
    kk = lax.broadcasted_iota(jnp.int32, (H, DH, H), 1)
    cc = lax.broadcasted_iota(jnp.int32, (H, DH, H), 2)
    p = (jj == HEADS * kk + cc // DH).astype(_f32)
    full = lax.dot_general(dsum, p, (((1,), (0,)), ((), ())),
                           preferred_element_type=_f32)
    out_ref[...] = full.reshape(NPAD, H)


def _expand_den(parts):
    return pl.pallas_call(
        _expand_body,
        grid=(1,),
        in_specs=[pl.BlockSpec((NC, 320, H), lambda i: (0, SBASE // 320, 0))],
        out_specs=pl.BlockSpec((NPAD, H), lambda i: (0, 0)),
        out_shape=jax.ShapeDtypeStruct((NPAD, H), _f32),
    )(parts)


def _ln(x, g, b, eps=1e-12):
    u = x.mean(-1, keepdims=True)
    s = ((x - u) ** 2).mean(-1, keepdims=True)
    return g * (x - u) / jnp.sqrt(s + eps) + b


def _post_body(parts_ref, dens_ref, x_ref, h_ref, wao_ref, bao_ref, g1_ref,
               b1_ref, wi_ref, bi_ref, wo_ref, bo_ref, g2_ref, b2_ref,
               wih_ref, whh_ref, bih_ref, bhh_ref, g3_ref, b3_ref,
               xo_ref, ho_ref):
    aggv = parts_ref[0] + parts_ref[1]
    den = dens_ref[...] + 1e-16
    agg = aggv / den
    x = x_ref[...]
    h = h_ref[...]
    attn = _ln(jnp.dot(agg, wao_ref[...].T, preferred_element_type=_f32)
               + bao_ref[...] + x, g1_ref[...], b1_ref[...])
    inter = jax.nn.gelu(jnp.dot(attn, wi_ref[...].T,
                                preferred_element_type=_f32) + bi_ref[...])
    m = _ln(jnp.dot(inter, wo_ref[...].T, preferred_element_type=_f32)
            + bo_ref[...] + attn, g2_ref[...], b2_ref[...])
    gi = jnp.dot(m, wih_ref[...].T, preferred_element_type=_f32) + bih_ref[...]
    gh = jnp.dot(h, whh_ref[...].T, preferred_element_type=_f32) + bhh_ref[...]
    r = jax.nn.sigmoid(gi[:, 0:H] + gh[:, 0:H])
    z = jax.nn.sigmoid(gi[:, H:2 * H] + gh[:, H:2 * H])
    n = jnp.tanh(gi[:, 2 * H:3 * H] + r * gh[:, 2 * H:3 * H])
    hn = (1.0 - z) * n + z * h
    ho_ref[...] = hn
    xo_ref[...] = _ln(hn, g3_ref[...], b3_ref[...])


def _post(parts, dens, x, h, Wao, bao, g1, b1, Wi, bi, Wo, bo, g2, b2,
          W_ih, W_hh, b_ih, b_hh, g3, b3):
    rows = N // 10
    full = lambda shape: pl.BlockSpec(shape, lambda i: (0,) * len(shape))
    vec = full((H,))
    return pl.pallas_call(
        _post_body,
        grid=(10,),
        in_specs=[
            pl.BlockSpec((NC, rows, H), lambda i: (0, i, 0)),
            pl.BlockSpec((rows, H), lambda i: (i, 0)),
            pl.BlockSpec((rows, H), lambda i: (i, 0)),
            pl.BlockSpec((rows, H), lambda i: (i, 0)),
            full((H, H)), vec,
            vec, vec,
            full((4 * H, H)), full((4 * H,)),
            full((H, 4 * H)), vec,
            vec, vec,
            full((3 * H, H)), full((3 * H, H)),
            full((3 * H,)), full((3 * H,)),
            vec, vec,
        ],
        out_specs=[
            pl.BlockSpec((rows, H), lambda i: (i, 0)),
            pl.BlockSpec((rows, H), lambda i: (i, 0)),
        ],
        out_shape=[
            jax.ShapeDtypeStruct((N, H), _f32),
            jax.ShapeDtypeStruct((N, H), _f32),
        ],
    )(parts, dens, x, h, Wao, bao, g1, b1, Wi, bi, Wo, bo, g2, b2,
      W_ih, W_hh, b_ih, b_hh, g3, b3)


def kernel(x, edge_index, edge_attr, Wq, bq, Wk, bk, Wv, bv, Wao, bao, g1, b1,
           Wi, bi, Wo, bo, g2, b2, W_ih, W_hh, b_ih, b_hh, g3, b3):
    src = edge_index[0]
    dst = edge_index[1]
    sk = jnp.sum(Wk, axis=1)
    sv = jnp.sum(Wv, axis=1)
    h = x
    for _ in range(T):
        qt, kv = _prep(x, Wq, bq, Wk, bk, Wv, bv)
        parts = _edge(qt, kv, src, dst, edge_attr, sk, sv)
        dens = _expand_den(parts)
        x, h = _post(parts, dens, x, h, Wao, bao, g1, b1, Wi, bi, Wo, bo,
                     g2, b2, W_ih, W_hh, b_ih, b_hh, g3, b3)
    return x
